# Initial kernel scaffold; baseline (speedup 1.0000x reference)
import jax
import jax.numpy as jnp
from jax.experimental import pallas as pl


def kernel(x_orig, edge_index_orig, x_anon, edge_index_anon, Wl1, bl1, Wr1, Wl2, bl2, Wr2):
    raise NotImplementedError("write your pallas kernel here")


# speedup vs baseline: 5.3284x; 5.3284x over previous
import functools

import jax
import jax.numpy as jnp
from jax import lax
from jax.experimental import pallas as pl
from jax.experimental.pallas import tpu as pltpu
from jax.experimental.pallas import tpu_sc as plsc

N = 10000
D = 128
E = 320000
WP = 144
NT = 16
C = 80
EPT = E // NT
NCHUNK = EPT // C
IB = 25
NB = NCHUNK // IB
RCH = N // C
ROWS_BLK = 2000


def _make_sc_agg(W):
  mesh = plsc.VectorSubcoreMesh(core_axis_name="c", subcore_axis_name="s")

  @functools.partial(
      pl.kernel,
      mesh=mesh,
      out_type=(
          jax.ShapeDtypeStruct((N, W), jnp.float32),
          jax.ShapeDtypeStruct((N, W), jnp.float32),
      ),
      scratch_types=[
          pltpu.VMEM((IB, C), jnp.int32),
          pltpu.VMEM((IB, C), jnp.int32),
          pltpu.VMEM((C, W), jnp.float32),
          pltpu.VMEM_SHARED((N, W), jnp.float32),
          pltpu.SemaphoreType.DMA,
      ],
      compiler_params=pltpu.CompilerParams(use_tc_tiling_on_sc=False),
  )
  def agg(x0, src0, dst0, x1, src1, dst1, out0, out1,
          sidx, didx, rows, acc, sem):
    cid = lax.axis_index("c")
    sid = lax.axis_index("s")

    z16 = jnp.zeros((16,), jnp.float32)

    def zrow(i, _):
      def zcol(j, _):
        rows[i, pl.ds(j * 16, 16)] = z16
        return 0
      return lax.fori_loop(0, W // 16, zcol, 0)
    lax.fori_loop(0, C, zrow, 0)

    n_mine = (RCH - sid + NT - 1) // NT

    def zchunk(k, _):
      r = sid + k * NT
      pltpu.sync_copy(rows, acc.at[pl.ds(r * C, C)])
      return 0
    lax.fori_loop(0, n_mine, zchunk, 0)

    plsc.subcore_barrier()

    def run(x, src, dst):
      def block(b, _):
        pltpu.sync_copy(src.at[sid, pl.ds(b * IB, IB)], sidx)
        pltpu.sync_copy(dst.at[sid, pl.ds(b * IB, IB)], didx)

        def chunk(k, _):
          pltpu.async_copy(x.at[sidx.at[k]], rows, sem).wait()
          pltpu.sync_copy(rows, acc.at[didx.at[k]], add=True)
          return 0
        lax.fori_loop(0, IB, chunk, 0)
        return 0
      lax.fori_loop(0, NB, block, 0)

    pl.when(cid == 0)(lambda: run(x0, src0, dst0))
    pl.when(cid == 1)(lambda: run(x1, src1, dst1))

    plsc.subcore_barrier()

    def wb(out):
      def wchunk(k, _):
        r = sid + k * NT
        pltpu.sync_copy(acc.at[pl.ds(r * C, C)], out.at[pl.ds(r * C, C)])
        return 0
      lax.fori_loop(0, n_mine, wchunk, 0)

    pl.when(cid == 0)(lambda: wb(out0))
    pl.when(cid == 1)(lambda: wb(out1))

  return agg


_sc_agg_wp = _make_sc_agg(WP)
_sc_agg_d = _make_sc_agg(D)


def _layer1_body(agg_ref, x_ref, wl_ref, bl_ref, wr_ref, o_ref):
  a = agg_ref[...]
  deg = jnp.maximum(a[:, D:D + 1], 1.0)
  mean = a[:, :D] / deg
  h = lax.dot_general(mean, wl_ref[...], (((1,), (1,)), ((), ())),
                      preferred_element_type=jnp.float32)
  h = h + bl_ref[...]
  h = h + lax.dot_general(x_ref[...], wr_ref[...], (((1,), (1,)), ((), ())),
                          preferred_element_type=jnp.float32)
  o_ref[...] = jnp.maximum(h, 0.0)


def _layer2_body(agg_ref, deg_ref, h_ref, wl_ref, bl_ref, wr_ref, o_ref):
  deg = jnp.maximum(deg_ref[...], 1.0)
  mean = agg_ref[...] / deg
  g = lax.dot_general(mean, wl_ref[...], (((1,), (1,)), ((), ())),
                      preferred_element_type=jnp.float32)
  g = g + bl_ref[...]
  g = g + lax.dot_general(h_ref[...], wr_ref[...], (((1,), (1,)), ((), ())),
                          preferred_element_type=jnp.float32)
  nrm = jnp.sqrt(jnp.sum(g * g, axis=1, keepdims=True))
  o_ref[...] = g / jnp.maximum(nrm, 1e-12)


def _tc_layer1(agg, x, wl, bl, wr):
  m = agg.shape[0]
  grid = m // ROWS_BLK
  return pl.pallas_call(
      _layer1_body,
      grid=(grid,),
      in_specs=[
          pl.BlockSpec((ROWS_BLK, WP), lambda i: (i, 0)),
          pl.BlockSpec((ROWS_BLK, D), lambda i: (i, 0)),
          pl.BlockSpec((D, D), lambda i: (0, 0)),
          pl.BlockSpec((1, D), lambda i: (0, 0)),
          pl.BlockSpec((D, D), lambda i: (0, 0)),
      ],
      out_specs=pl.BlockSpec((ROWS_BLK, D), lambda i: (i, 0)),
      out_shape=jax.ShapeDtypeStruct((m, D), jnp.float32),
  )(agg, x, wl, bl, wr)


def _tc_layer2(agg, deg, h, wl, bl, wr):
  m = agg.shape[0]
  grid = m // ROWS_BLK
  return pl.pallas_call(
      _layer2_body,
      grid=(grid,),
      in_specs=[
          pl.BlockSpec((ROWS_BLK, D), lambda i: (i, 0)),
          pl.BlockSpec((ROWS_BLK, 1), lambda i: (i, 0)),
          pl.BlockSpec((ROWS_BLK, D), lambda i: (i, 0)),
          pl.BlockSpec((D, D), lambda i: (0, 0)),
          pl.BlockSpec((1, D), lambda i: (0, 0)),
          pl.BlockSpec((D, D), lambda i: (0, 0)),
      ],
      out_specs=pl.BlockSpec((ROWS_BLK, D), lambda i: (i, 0)),
      out_shape=jax.ShapeDtypeStruct((m, D), jnp.float32),
  )(agg, deg, h, wl, bl, wr)


def kernel(x_orig, edge_index_orig, x_anon, edge_index_anon,
           Wl1, bl1, Wr1, Wl2, bl2, Wr2):
  src_o = edge_index_orig[0].astype(jnp.int32).reshape(NT, NCHUNK, C)
  dst_o = edge_index_orig[1].astype(jnp.int32).reshape(NT, NCHUNK, C)
  src_a = edge_index_anon[0].astype(jnp.int32).reshape(NT, NCHUNK, C)
  dst_a = edge_index_anon[1].astype(jnp.int32).reshape(NT, NCHUNK, C)

  pad = jnp.concatenate(
      [jnp.ones((N, 1), jnp.float32), jnp.zeros((N, WP - D - 1), jnp.float32)],
      axis=1)
  xp_o = jnp.concatenate([x_orig, pad], axis=1)
  xp_a = jnp.concatenate([x_anon, pad], axis=1)

  agg1_o, agg1_a = _sc_agg_wp(xp_o, src_o, dst_o, xp_a, src_a, dst_a)
  agg1 = jnp.concatenate([agg1_o, agg1_a], axis=0)
  x2 = jnp.concatenate([x_orig, x_anon], axis=0)

  h = _tc_layer1(agg1, x2, Wl1, bl1.reshape(1, D), Wr1)

  h_o = h[:N]
  h_a = h[N:]
  agg2_o, agg2_a = _sc_agg_d(h_o, src_o, dst_o, h_a, src_a, dst_a)
  agg2 = jnp.concatenate([agg2_o, agg2_a], axis=0)
  deg = lax.slice(agg1, (0, D), (2 * N, D + 1))

  out = _tc_layer2(agg2, deg, h, Wl2, bl2.reshape(1, D), Wr2)
  return (out[:N], out[N:])


# speedup vs baseline: 8.8035x; 1.6522x over previous
import functools

import jax
import jax.numpy as jnp
from jax import lax
from jax.experimental import pallas as pl
from jax.experimental.pallas import tpu as pltpu
from jax.experimental.pallas import tpu_sc as plsc

N = 10000
D = 128
E = 320000
WP = 144
NT = 16
C = 80
EPT = E // NT
IB = 50
NB = EPT // (IB * C)
RCH = N // C
ROWS_BLK = 2000


def _make_sc_agg(W):
  mesh = plsc.VectorSubcoreMesh(core_axis_name="c", subcore_axis_name="s")

  @functools.partial(
      pl.kernel,
      mesh=mesh,
      out_type=jax.ShapeDtypeStruct((2 * N, W), jnp.float32),
      scratch_types=[
          pltpu.VMEM((IB, C), jnp.int32),
          pltpu.VMEM((IB, C), jnp.int32),
          pltpu.VMEM((C, W), jnp.float32),
          pltpu.VMEM((C, W), jnp.float32),
          pltpu.VMEM_SHARED((N, W), jnp.float32),
          pltpu.SemaphoreType.DMA,
          pltpu.SemaphoreType.DMA,
      ],
      compiler_params=pltpu.CompilerParams(use_tc_tiling_on_sc=False),
  )
  def agg(x, src0, dst0, src1, dst1, out,
          sidx, didx, rows0, rows1, acc, sem0, sem1):
    cid = lax.axis_index("c")
    sid = lax.axis_index("s")

    z16 = jnp.zeros((16,), jnp.float32)

    def zrow(i, _):
      def zcol(j, _):
        rows0[i, pl.ds(j * 16, 16)] = z16
        return 0
      return lax.fori_loop(0, W // 16, zcol, 0)
    lax.fori_loop(0, C, zrow, 0)

    n_mine = (RCH - sid + NT - 1) // NT

    def zchunk(k, _):
      r = sid + k * NT
      pltpu.sync_copy(rows0, acc.at[pl.ds(r * C, C)])
      return 0
    lax.fori_loop(0, n_mine, zchunk, 0)

    plsc.subcore_barrier()

    def run(src, dst):
      def block(b, _):
        pltpu.sync_copy(src.at[sid, pl.ds(b * IB, IB)], sidx)
        pltpu.sync_copy(dst.at[sid, pl.ds(b * IB, IB)], didx)
        pltpu.async_copy(x.at[sidx.at[0]], rows0, sem0)

        def pair(k, _):
          c0 = 2 * k
          c1 = 2 * k + 1
          pltpu.async_copy(x.at[sidx.at[c1]], rows1, sem1)
          pltpu.make_async_copy(x.at[sidx.at[c0]], rows0, sem0).wait()
          pltpu.sync_copy(rows0, acc.at[didx.at[c0]], add=True)

          @pl.when(c0 + 2 < IB)
          def _():
            pltpu.async_copy(x.at[sidx.at[c0 + 2]], rows0, sem0)

          pltpu.make_async_copy(x.at[sidx.at[c1]], rows1, sem1).wait()
          pltpu.sync_copy(rows1, acc.at[didx.at[c1]], add=True)
          return 0
        lax.fori_loop(0, IB // 2, pair, 0)
        return 0
      lax.fori_loop(0, NB, block, 0)

    pl.when(cid == 0)(lambda: run(src0, dst0))
    pl.when(cid == 1)(lambda: run(src1, dst1))

    plsc.subcore_barrier()

    def wchunk(k, _):
      r = sid + k * NT
      pltpu.sync_copy(acc.at[pl.ds(r * C, C)],
                      out.at[pl.ds(cid * N + r * C, C)])
      return 0
    lax.fori_loop(0, n_mine, wchunk, 0)

  return agg


_sc_agg_wp = _make_sc_agg(WP)
_sc_agg_d = _make_sc_agg(D)


def _layer1_body(agg_ref, x_ref, wl_ref, bl_ref, wr_ref, o_ref):
  a = agg_ref[...]
  deg = jnp.maximum(a[:, D:D + 1], 1.0)
  mean = a[:, :D] / deg
  h = lax.dot_general(mean, wl_ref[...], (((1,), (1,)), ((), ())),
                      preferred_element_type=jnp.float32)
  h = h + bl_ref[...]
  h = h + lax.dot_general(x_ref[:, :D], wr_ref[...], (((1,), (1,)), ((), ())),
                          preferred_element_type=jnp.float32)
  o_ref[...] = jnp.maximum(h, 0.0)


def _layer2_body(agg_ref, deg_ref, h_ref, wl_ref, bl_ref, wr_ref, o_ref):
  deg = jnp.maximum(deg_ref[...], 1.0)
  mean = agg_ref[...] / deg
  g = lax.dot_general(mean, wl_ref[...], (((1,), (1,)), ((), ())),
                      preferred_element_type=jnp.float32)
  g = g + bl_ref[...]
  g = g + lax.dot_general(h_ref[...], wr_ref[...], (((1,), (1,)), ((), ())),
                          preferred_element_type=jnp.float32)
  nrm = jnp.sqrt(jnp.sum(g * g, axis=1, keepdims=True))
  o_ref[...] = g / jnp.maximum(nrm, 1e-12)


def _tc_layer1(agg, xp, wl, bl, wr):
  m = agg.shape[0]
  grid = m // ROWS_BLK
  return pl.pallas_call(
      _layer1_body,
      grid=(grid,),
      in_specs=[
          pl.BlockSpec((ROWS_BLK, WP), lambda i: (i, 0)),
          pl.BlockSpec((ROWS_BLK, WP), lambda i: (i, 0)),
          pl.BlockSpec((D, D), lambda i: (0, 0)),
          pl.BlockSpec((1, D), lambda i: (0, 0)),
          pl.BlockSpec((D, D), lambda i: (0, 0)),
      ],
      out_specs=pl.BlockSpec((ROWS_BLK, D), lambda i: (i, 0)),
      out_shape=jax.ShapeDtypeStruct((m, D), jnp.float32),
  )(agg, xp, wl, bl, wr)


def _tc_layer2(agg, deg, h, wl, bl, wr):
  m = agg.shape[0]
  grid = m // ROWS_BLK
  return pl.pallas_call(
      _layer2_body,
      grid=(grid,),
      in_specs=[
          pl.BlockSpec((ROWS_BLK, D), lambda i: (i, 0)),
          pl.BlockSpec((ROWS_BLK, 1), lambda i: (i, 0)),
          pl.BlockSpec((ROWS_BLK, D), lambda i: (i, 0)),
          pl.BlockSpec((D, D), lambda i: (0, 0)),
          pl.BlockSpec((1, D), lambda i: (0, 0)),
          pl.BlockSpec((D, D), lambda i: (0, 0)),
      ],
      out_specs=pl.BlockSpec((ROWS_BLK, D), lambda i: (i, 0)),
      out_shape=jax.ShapeDtypeStruct((m, D), jnp.float32),
  )(agg, deg, h, wl, bl, wr)


def kernel(x_orig, edge_index_orig, x_anon, edge_index_anon,
           Wl1, bl1, Wr1, Wl2, bl2, Wr2):
  src_o = edge_index_orig[0].astype(jnp.int32).reshape(NT, NB * IB, C)
  dst_o = edge_index_orig[1].astype(jnp.int32).reshape(NT, NB * IB, C)
  src_a = (edge_index_anon[0].astype(jnp.int32) + N).reshape(NT, NB * IB, C)
  dst_a = edge_index_anon[1].astype(jnp.int32).reshape(NT, NB * IB, C)

  x2 = jnp.concatenate([x_orig, x_anon], axis=0)
  pad = jnp.concatenate(
      [jnp.ones((2 * N, 1), jnp.float32),
       jnp.zeros((2 * N, WP - D - 1), jnp.float32)], axis=1)
  xp = jnp.concatenate([x2, pad], axis=1)

  agg1 = _sc_agg_wp(xp, src_o, dst_o, src_a, dst_a)
  h = _tc_layer1(agg1, xp, Wl1, bl1.reshape(1, D), Wr1)
  agg2 = _sc_agg_d(h, src_o, dst_o, src_a, dst_a)
  deg = lax.slice(agg1, (0, D), (2 * N, D + 1))
  out = _tc_layer2(agg2, deg, h, Wl2, bl2.reshape(1, D), Wr2)
  return (out[:N], out[N:])


# speedup vs baseline: 9.0746x; 1.0308x over previous
import functools

import jax
import jax.numpy as jnp
from jax import lax
from jax.experimental import pallas as pl
from jax.experimental.pallas import tpu as pltpu
from jax.experimental.pallas import tpu_sc as plsc

N = 10000
D = 128
E = 320000
WP = 144
NT = 16
C = 80
EPT = E // NT
NCHUNK = EPT // C
IB = 10
NB = NCHUNK // IB
RCH = N // C
ROWS_BLK = 2000


def _make_sc_agg(W):
  mesh = plsc.VectorSubcoreMesh(core_axis_name="c", subcore_axis_name="s")

  @functools.partial(
      pl.kernel,
      mesh=mesh,
      out_type=jax.ShapeDtypeStruct((2 * N, W), jnp.float32),
      scratch_types=[
          pltpu.VMEM((2, IB, C), jnp.int32),
          pltpu.VMEM((2, IB, C), jnp.int32),
          pltpu.VMEM((C, W), jnp.float32),
          pltpu.VMEM((C, W), jnp.float32),
          pltpu.VMEM_SHARED((N, W), jnp.float32),
          pltpu.SemaphoreType.DMA,
          pltpu.SemaphoreType.DMA,
          pltpu.SemaphoreType.DMA,
          pltpu.SemaphoreType.DMA,
          pltpu.SemaphoreType.DMA,
          pltpu.SemaphoreType.DMA,
          pltpu.SemaphoreType.DMA,
      ],
      compiler_params=pltpu.CompilerParams(use_tc_tiling_on_sc=False),
  )
  def agg(x, src0, dst0, src1, dst1, out,
          sidx, didx, rows0, rows1, acc,
          gsem0, gsem1, ssem0, ssem1, sisem, disem, wsem):
    cid = lax.axis_index("c")
    sid = lax.axis_index("s")

    z16 = jnp.zeros((16,), jnp.float32)

    def zrow(i, _):
      def zcol(j, _):
        rows0[i, pl.ds(j * 16, 16)] = z16
        return 0
      return lax.fori_loop(0, W // 16, zcol, 0)
    lax.fori_loop(0, C, zrow, 0)

    n_mine = (RCH - sid + NT - 1) // NT

    def zchunk(k, _):
      r = sid + k * NT
      pltpu.async_copy(rows0, acc.at[pl.ds(r * C, C)], wsem)
      return 0
    lax.fori_loop(0, n_mine, zchunk, 0)

    def zdrain(k, _):
      pltpu.make_async_copy(rows0, acc.at[pl.ds(sid * C, C)], wsem).wait()
      return 0
    lax.fori_loop(0, n_mine, zdrain, 0)

    plsc.subcore_barrier()

    rows = (rows0, rows1)
    gsem = (gsem0, gsem1)
    ssem = (ssem0, ssem1)

    def run(src, dst):
      pltpu.async_copy(src.at[sid, pl.ds(0, IB)], sidx.at[0], sisem)
      pltpu.async_copy(dst.at[sid, pl.ds(0, IB)], didx.at[0], disem)

      def block(b, _):
        par = lax.rem(b, 2)
        par2 = lax.rem(b + 1, 2)
        pltpu.make_async_copy(src.at[sid, pl.ds(b * IB, IB)],
                              sidx.at[par], sisem).wait()
        pltpu.make_async_copy(dst.at[sid, pl.ds(b * IB, IB)],
                              didx.at[par], disem).wait()

        for rem in range(IB):
          buf = rem % 2
          if rem >= 2:
            pltpu.make_async_copy(
                rows[buf], acc.at[didx.at[par, rem - 2]], ssem[buf]).wait()
          else:
            @pl.when(b > 0)
            def _(buf=buf, rem=rem, par2=par2):
              pltpu.make_async_copy(
                  rows[buf], acc.at[didx.at[par2, IB + rem - 2]],
                  ssem[buf]).wait()
          pltpu.async_copy(x.at[sidx.at[par, rem]], rows[buf], gsem[buf])

          if rem == 3:
            @pl.when(b + 1 < NB)
            def _(par2=par2):
              pltpu.async_copy(src.at[sid, pl.ds((b + 1) * IB, IB)],
                               sidx.at[par2], sisem)
              pltpu.async_copy(dst.at[sid, pl.ds((b + 1) * IB, IB)],
                               didx.at[par2], disem)

          jbuf = 1 - buf
          if rem >= 1:
            pltpu.make_async_copy(
                x.at[sidx.at[par, rem - 1]], rows[jbuf], gsem[jbuf]).wait()
            pltpu.async_copy(rows[jbuf], acc.at[didx.at[par, rem - 1]],
                             ssem[jbuf], add=True)
          else:
            @pl.when(b > 0)
            def _(jbuf=jbuf, par2=par2):
              pltpu.make_async_copy(
                  x.at[sidx.at[par2, IB - 1]], rows[jbuf],
                  gsem[jbuf]).wait()
              pltpu.async_copy(rows[jbuf], acc.at[didx.at[par2, IB - 1]],
                               ssem[jbuf], add=True)
        return 0
      lax.fori_loop(0, NB, block, 0)

      lpar = (NB - 1) % 2
      pltpu.make_async_copy(x.at[sidx.at[lpar, IB - 1]], rows1,
                            gsem1).wait()
      pltpu.async_copy(rows1, acc.at[didx.at[lpar, IB - 1]], ssem1,
                       add=True)
      pltpu.make_async_copy(rows0, acc.at[didx.at[lpar, IB - 2]],
                            ssem0).wait()
      pltpu.make_async_copy(rows1, acc.at[didx.at[lpar, IB - 1]],
                            ssem1).wait()

    pl.when(cid == 0)(lambda: run(src0, dst0))
    pl.when(cid == 1)(lambda: run(src1, dst1))

    plsc.subcore_barrier()

    def wchunk(k, _):
      r = sid + k * NT
      pltpu.async_copy(acc.at[pl.ds(r * C, C)],
                       out.at[pl.ds(cid * N + r * C, C)], wsem)
      return 0
    lax.fori_loop(0, n_mine, wchunk, 0)

    def wdrain(k, _):
      pltpu.make_async_copy(acc.at[pl.ds(sid * C, C)],
                            out.at[pl.ds(cid * N + sid * C, C)], wsem).wait()
      return 0
    lax.fori_loop(0, n_mine, wdrain, 0)

  return agg


_sc_agg_wp = _make_sc_agg(WP)
_sc_agg_d = _make_sc_agg(D)


def _layer1_body(agg_ref, x_ref, wl_ref, bl_ref, wr_ref, o_ref):
  a = agg_ref[...]
  deg = jnp.maximum(a[:, D:D + 1], 1.0)
  mean = a[:, :D] / deg
  h = lax.dot_general(mean, wl_ref[...], (((1,), (1,)), ((), ())),
                      preferred_element_type=jnp.float32)
  h = h + bl_ref[...]
  h = h + lax.dot_general(x_ref[:, :D], wr_ref[...], (((1,), (1,)), ((), ())),
                          preferred_element_type=jnp.float32)
  o_ref[...] = jnp.maximum(h, 0.0)


def _layer2_body(agg_ref, deg_ref, h_ref, wl_ref, bl_ref, wr_ref, o_ref):
  deg = jnp.maximum(deg_ref[...], 1.0)
  mean = agg_ref[...] / deg
  g = lax.dot_general(mean, wl_ref[...], (((1,), (1,)), ((), ())),
                      preferred_element_type=jnp.float32)
  g = g + bl_ref[...]
  g = g + lax.dot_general(h_ref[...], wr_ref[...], (((1,), (1,)), ((), ())),
                          preferred_element_type=jnp.float32)
  nrm = jnp.sqrt(jnp.sum(g * g, axis=1, keepdims=True))
  o_ref[...] = g / jnp.maximum(nrm, 1e-12)


def _tc_layer1(agg, xp, wl, bl, wr):
  m = agg.shape[0]
  grid = m // ROWS_BLK
  return pl.pallas_call(
      _layer1_body,
      grid=(grid,),
      in_specs=[
          pl.BlockSpec((ROWS_BLK, WP), lambda i: (i, 0)),
          pl.BlockSpec((ROWS_BLK, WP), lambda i: (i, 0)),
          pl.BlockSpec((D, D), lambda i: (0, 0)),
          pl.BlockSpec((1, D), lambda i: (0, 0)),
          pl.BlockSpec((D, D), lambda i: (0, 0)),
      ],
      out_specs=pl.BlockSpec((ROWS_BLK, D), lambda i: (i, 0)),
      out_shape=jax.ShapeDtypeStruct((m, D), jnp.float32),
  )(agg, xp, wl, bl, wr)


def _tc_layer2(agg, deg, h, wl, bl, wr):
  m = agg.shape[0]
  grid = m // ROWS_BLK
  return pl.pallas_call(
      _layer2_body,
      grid=(grid,),
      in_specs=[
          pl.BlockSpec((ROWS_BLK, D), lambda i: (i, 0)),
          pl.BlockSpec((ROWS_BLK, 1), lambda i: (i, 0)),
          pl.BlockSpec((ROWS_BLK, D), lambda i: (i, 0)),
          pl.BlockSpec((D, D), lambda i: (0, 0)),
          pl.BlockSpec((1, D), lambda i: (0, 0)),
          pl.BlockSpec((D, D), lambda i: (0, 0)),
      ],
      out_specs=pl.BlockSpec((ROWS_BLK, D), lambda i: (i, 0)),
      out_shape=jax.ShapeDtypeStruct((m, D), jnp.float32),
  )(agg, deg, h, wl, bl, wr)


def kernel(x_orig, edge_index_orig, x_anon, edge_index_anon,
           Wl1, bl1, Wr1, Wl2, bl2, Wr2):
  src_o = edge_index_orig[0].astype(jnp.int32).reshape(NT, NB * IB, C)
  dst_o = edge_index_orig[1].astype(jnp.int32).reshape(NT, NB * IB, C)
  src_a = (edge_index_anon[0].astype(jnp.int32) + N).reshape(NT, NB * IB, C)
  dst_a = edge_index_anon[1].astype(jnp.int32).reshape(NT, NB * IB, C)

  x2 = jnp.concatenate([x_orig, x_anon], axis=0)
  pad = jnp.concatenate(
      [jnp.ones((2 * N, 1), jnp.float32),
       jnp.zeros((2 * N, WP - D - 1), jnp.float32)], axis=1)
  xp = jnp.concatenate([x2, pad], axis=1)

  agg1 = _sc_agg_wp(xp, src_o, dst_o, src_a, dst_a)
  h = _tc_layer1(agg1, xp, Wl1, bl1.reshape(1, D), Wr1)
  agg2 = _sc_agg_d(h, src_o, dst_o, src_a, dst_a)
  deg = lax.slice(agg1, (0, D), (2 * N, D + 1))
  out = _tc_layer2(agg2, deg, h, Wl2, bl2.reshape(1, D), Wr2)
  return (out[:N], out[N:])


# speedup vs baseline: 9.3107x; 1.0260x over previous
import functools

import jax
import jax.numpy as jnp
from jax import lax
from jax.experimental import pallas as pl
from jax.experimental.pallas import tpu as pltpu
from jax.experimental.pallas import tpu_sc as plsc

N = 10000
D = 128
E = 320000
WP = 144
NT = 16
C = 80
EPT = E // NT
NCHUNK = EPT // C
IB = 10
NB = NCHUNK // IB
RCH = N // C
ROWS_BLK = 2000


def _make_sc_agg(W, split_deg=False):
  mesh = plsc.VectorSubcoreMesh(core_axis_name="c", subcore_axis_name="s")

  if split_deg:
    out_type = (jax.ShapeDtypeStruct((2 * N, D), jnp.float32),
                jax.ShapeDtypeStruct((2 * N, W - D), jnp.float32))
  else:
    out_type = jax.ShapeDtypeStruct((2 * N, W), jnp.float32)

  @functools.partial(
      pl.kernel,
      mesh=mesh,
      out_type=out_type,
      scratch_types=[
          pltpu.VMEM((2, IB, C), jnp.int32),
          pltpu.VMEM((2, IB, C), jnp.int32),
          pltpu.VMEM((C, W), jnp.float32),
          pltpu.VMEM((C, W), jnp.float32),
          pltpu.VMEM_SHARED((N, W), jnp.float32),
          pltpu.SemaphoreType.DMA,
          pltpu.SemaphoreType.DMA,
          pltpu.SemaphoreType.DMA,
          pltpu.SemaphoreType.DMA,
          pltpu.SemaphoreType.DMA,
          pltpu.SemaphoreType.DMA,
          pltpu.SemaphoreType.DMA,
      ],
      compiler_params=pltpu.CompilerParams(use_tc_tiling_on_sc=False),
  )
  def agg(x, src0, dst0, src1, dst1, *rest):
    if split_deg:
      (out, dout, sidx, didx, rows0, rows1, acc,
       gsem0, gsem1, ssem0, ssem1, sisem, disem, wsem) = rest
    else:
      (out, sidx, didx, rows0, rows1, acc,
       gsem0, gsem1, ssem0, ssem1, sisem, disem, wsem) = rest
    cid = lax.axis_index("c")
    sid = lax.axis_index("s")

    z16 = jnp.zeros((16,), jnp.float32)

    def zrow(i, _):
      def zcol(j, _):
        rows0[i, pl.ds(j * 16, 16)] = z16
        return 0
      return lax.fori_loop(0, W // 16, zcol, 0)
    lax.fori_loop(0, C, zrow, 0)

    n_mine = (RCH - sid + NT - 1) // NT

    def zchunk(k, _):
      r = sid + k * NT
      pltpu.async_copy(rows0, acc.at[pl.ds(r * C, C)], wsem)
      return 0
    lax.fori_loop(0, n_mine, zchunk, 0)

    def zdrain(k, _):
      pltpu.make_async_copy(rows0, acc.at[pl.ds(sid * C, C)], wsem).wait()
      return 0
    lax.fori_loop(0, n_mine, zdrain, 0)

    plsc.subcore_barrier()

    rows = (rows0, rows1)
    gsem = (gsem0, gsem1)
    ssem = (ssem0, ssem1)

    def run(src, dst):
      pltpu.async_copy(src.at[sid, pl.ds(0, IB)], sidx.at[0], sisem)
      pltpu.async_copy(dst.at[sid, pl.ds(0, IB)], didx.at[0], disem)

      def block(b, _):
        par = lax.rem(b, 2)
        par2 = lax.rem(b + 1, 2)
        pltpu.make_async_copy(src.at[sid, pl.ds(b * IB, IB)],
                              sidx.at[par], sisem).wait()
        pltpu.make_async_copy(dst.at[sid, pl.ds(b * IB, IB)],
                              didx.at[par], disem).wait()

        for rem in range(IB):
          buf = rem % 2
          if rem >= 2:
            pltpu.make_async_copy(
                rows[buf], acc.at[didx.at[par, rem - 2]], ssem[buf]).wait()
          else:
            @pl.when(b > 0)
            def _(buf=buf, rem=rem, par2=par2):
              pltpu.make_async_copy(
                  rows[buf], acc.at[didx.at[par2, IB + rem - 2]],
                  ssem[buf]).wait()
          pltpu.async_copy(x.at[sidx.at[par, rem]], rows[buf], gsem[buf])

          if rem == 3:
            @pl.when(b + 1 < NB)
            def _(par2=par2):
              pltpu.async_copy(src.at[sid, pl.ds((b + 1) * IB, IB)],
                               sidx.at[par2], sisem)
              pltpu.async_copy(dst.at[sid, pl.ds((b + 1) * IB, IB)],
                               didx.at[par2], disem)

          jbuf = 1 - buf
          if rem >= 1:
            pltpu.make_async_copy(
                x.at[sidx.at[par, rem - 1]], rows[jbuf], gsem[jbuf]).wait()
            pltpu.async_copy(rows[jbuf], acc.at[didx.at[par, rem - 1]],
                             ssem[jbuf], add=True)
          else:
            @pl.when(b > 0)
            def _(jbuf=jbuf, par2=par2):
              pltpu.make_async_copy(
                  x.at[sidx.at[par2, IB - 1]], rows[jbuf],
                  gsem[jbuf]).wait()
              pltpu.async_copy(rows[jbuf], acc.at[didx.at[par2, IB - 1]],
                               ssem[jbuf], add=True)
        return 0
      lax.fori_loop(0, NB, block, 0)

      lpar = (NB - 1) % 2
      pltpu.make_async_copy(x.at[sidx.at[lpar, IB - 1]], rows1,
                            gsem1).wait()
      pltpu.async_copy(rows1, acc.at[didx.at[lpar, IB - 1]], ssem1,
                       add=True)
      pltpu.make_async_copy(rows0, acc.at[didx.at[lpar, IB - 2]],
                            ssem0).wait()
      pltpu.make_async_copy(rows1, acc.at[didx.at[lpar, IB - 1]],
                            ssem1).wait()

    pl.when(cid == 0)(lambda: run(src0, dst0))
    pl.when(cid == 1)(lambda: run(src1, dst1))

    plsc.subcore_barrier()

    if split_deg:
      def wchunk(k, _):
        r = sid + k * NT
        pltpu.async_copy(acc.at[pl.ds(r * C, C), pl.ds(0, D)],
                         out.at[pl.ds(cid * N + r * C, C)], wsem)
        pltpu.async_copy(acc.at[pl.ds(r * C, C), pl.ds(D, W - D)],
                         dout.at[pl.ds(cid * N + r * C, C)], wsem)
        return 0
      lax.fori_loop(0, n_mine, wchunk, 0)

      def wdrain(k, _):
        pltpu.make_async_copy(acc.at[pl.ds(sid * C, C), pl.ds(0, D)],
                              out.at[pl.ds(cid * N + sid * C, C)],
                              wsem).wait()
        pltpu.make_async_copy(acc.at[pl.ds(sid * C, C), pl.ds(D, W - D)],
                              dout.at[pl.ds(cid * N + sid * C, C)],
                              wsem).wait()
        return 0
      lax.fori_loop(0, n_mine, wdrain, 0)
    else:
      def wchunk(k, _):
        r = sid + k * NT
        pltpu.async_copy(acc.at[pl.ds(r * C, C)],
                         out.at[pl.ds(cid * N + r * C, C)], wsem)
        return 0
      lax.fori_loop(0, n_mine, wchunk, 0)

      def wdrain(k, _):
        pltpu.make_async_copy(acc.at[pl.ds(sid * C, C)],
                              out.at[pl.ds(cid * N + sid * C, C)], wsem).wait()
        return 0
      lax.fori_loop(0, n_mine, wdrain, 0)

  return agg


_sc_agg_wp = _make_sc_agg(WP, split_deg=True)
_sc_agg_d = _make_sc_agg(D)


def _layer1_body(agg_ref, deg_ref, x_ref, wl_ref, bl_ref, wr_ref, o_ref):
  deg = jnp.maximum(deg_ref[...], 1.0)
  mean = agg_ref[...] / deg
  h = lax.dot_general(mean, wl_ref[...], (((1,), (1,)), ((), ())),
                      preferred_element_type=jnp.float32)
  h = h + bl_ref[...]
  h = h + lax.dot_general(x_ref[:, :D], wr_ref[...], (((1,), (1,)), ((), ())),
                          preferred_element_type=jnp.float32)
  o_ref[...] = jnp.maximum(h, 0.0)


def _layer2_body(agg_ref, deg_ref, h_ref, wl_ref, bl_ref, wr_ref, o_ref):
  deg = jnp.maximum(deg_ref[...], 1.0)
  mean = agg_ref[...] / deg
  g = lax.dot_general(mean, wl_ref[...], (((1,), (1,)), ((), ())),
                      preferred_element_type=jnp.float32)
  g = g + bl_ref[...]
  g = g + lax.dot_general(h_ref[...], wr_ref[...], (((1,), (1,)), ((), ())),
                          preferred_element_type=jnp.float32)
  nrm = jnp.sqrt(jnp.sum(g * g, axis=1, keepdims=True))
  o_ref[...] = g / jnp.maximum(nrm, 1e-12)


def _tc_layer1(agg, deg, xp, wl, bl, wr):
  m = agg.shape[0]
  grid = m // ROWS_BLK
  return pl.pallas_call(
      _layer1_body,
      grid=(grid,),
      in_specs=[
          pl.BlockSpec((ROWS_BLK, D), lambda i: (i, 0)),
          pl.BlockSpec((ROWS_BLK, 1), lambda i: (i, 0)),
          pl.BlockSpec((ROWS_BLK, WP), lambda i: (i, 0)),
          pl.BlockSpec((D, D), lambda i: (0, 0)),
          pl.BlockSpec((1, D), lambda i: (0, 0)),
          pl.BlockSpec((D, D), lambda i: (0, 0)),
      ],
      out_specs=pl.BlockSpec((ROWS_BLK, D), lambda i: (i, 0)),
      out_shape=jax.ShapeDtypeStruct((m, D), jnp.float32),
  )(agg, deg, xp, wl, bl, wr)


def _tc_layer2(agg, deg, h, wl, bl, wr):
  m = agg.shape[0]
  grid = m // ROWS_BLK
  return pl.pallas_call(
      _layer2_body,
      grid=(grid,),
      in_specs=[
          pl.BlockSpec((ROWS_BLK, D), lambda i: (i, 0)),
          pl.BlockSpec((ROWS_BLK, 1), lambda i: (i, 0)),
          pl.BlockSpec((ROWS_BLK, D), lambda i: (i, 0)),
          pl.BlockSpec((D, D), lambda i: (0, 0)),
          pl.BlockSpec((1, D), lambda i: (0, 0)),
          pl.BlockSpec((D, D), lambda i: (0, 0)),
      ],
      out_specs=pl.BlockSpec((ROWS_BLK, D), lambda i: (i, 0)),
      out_shape=jax.ShapeDtypeStruct((m, D), jnp.float32),
  )(agg, deg, h, wl, bl, wr)


def kernel(x_orig, edge_index_orig, x_anon, edge_index_anon,
           Wl1, bl1, Wr1, Wl2, bl2, Wr2):
  src_o = edge_index_orig[0].astype(jnp.int32).reshape(NT, NB * IB, C)
  dst_o = edge_index_orig[1].astype(jnp.int32).reshape(NT, NB * IB, C)
  src_a = (edge_index_anon[0].astype(jnp.int32) + N).reshape(NT, NB * IB, C)
  dst_a = edge_index_anon[1].astype(jnp.int32).reshape(NT, NB * IB, C)

  x2 = jnp.concatenate([x_orig, x_anon], axis=0)
  pad = jnp.concatenate(
      [jnp.ones((2 * N, 1), jnp.float32),
       jnp.zeros((2 * N, WP - D - 1), jnp.float32)], axis=1)
  xp = jnp.concatenate([x2, pad], axis=1)

  agg1, degw = _sc_agg_wp(xp, src_o, dst_o, src_a, dst_a)
  deg = lax.slice(degw, (0, 0), (2 * N, 1))
  h = _tc_layer1(agg1, deg, xp, Wl1, bl1.reshape(1, D), Wr1)
  agg2 = _sc_agg_d(h, src_o, dst_o, src_a, dst_a)
  out = _tc_layer2(agg2, deg, h, Wl2, bl2.reshape(1, D), Wr2)
  return (out[:N], out[N:])


# speedup vs baseline: 9.9549x; 1.0692x over previous
import functools

import jax
import jax.numpy as jnp
from jax import lax
from jax.experimental import pallas as pl
from jax.experimental.pallas import tpu as pltpu
from jax.experimental.pallas import tpu_sc as plsc

N = 10000
D = 128
E = 320000
WP = 144
NT = 16
C = 80
EPT = E // NT
NCHUNK = EPT // C
IB = 10
NB = NCHUNK // IB
RCH = N // C
ROWS_BLK = 2000


def _make_sc_agg(W, split_deg=False):
  mesh = plsc.VectorSubcoreMesh(core_axis_name="c", subcore_axis_name="s")

  DW = 16
  if split_deg:
    out_type = (jax.ShapeDtypeStruct((2 * N, W), jnp.float32),
                jax.ShapeDtypeStruct((2 * N, DW), jnp.float32))
  else:
    out_type = jax.ShapeDtypeStruct((2 * N, W), jnp.float32)

  @functools.partial(
      pl.kernel,
      mesh=mesh,
      out_type=out_type,
      scratch_types=[
          pltpu.VMEM((2, IB, C), jnp.int32),
          pltpu.VMEM((2, IB, C), jnp.int32),
          pltpu.VMEM((C, W), jnp.float32),
          pltpu.VMEM((C, W), jnp.float32),
          pltpu.VMEM_SHARED((N, W), jnp.float32),
          pltpu.SemaphoreType.DMA,
          pltpu.SemaphoreType.DMA,
          pltpu.SemaphoreType.DMA,
          pltpu.SemaphoreType.DMA,
          pltpu.SemaphoreType.DMA,
          pltpu.SemaphoreType.DMA,
          pltpu.SemaphoreType.DMA,
      ] + ([
          pltpu.VMEM((C, DW), jnp.float32),
          pltpu.VMEM_SHARED((N, DW), jnp.float32),
          pltpu.SemaphoreType.DMA,
          pltpu.SemaphoreType.DMA,
      ] if split_deg else []),
      compiler_params=pltpu.CompilerParams(use_tc_tiling_on_sc=False),
  )
  def agg(x, src0, dst0, src1, dst1, *rest):
    if split_deg:
      (out, dout, sidx, didx, rows0, rows1, acc,
       gsem0, gsem1, ssem0, ssem1, sisem, disem, wsem,
       ones_buf, acc_deg, dsem0, dsem1) = rest
      dsem = (dsem0, dsem1)
    else:
      (out, sidx, didx, rows0, rows1, acc,
       gsem0, gsem1, ssem0, ssem1, sisem, disem, wsem) = rest
    cid = lax.axis_index("c")
    sid = lax.axis_index("s")

    z16 = jnp.zeros((16,), jnp.float32)

    def zrow(i, _):
      def zcol(j, _):
        rows0[i, pl.ds(j * 16, 16)] = z16
        return 0
      return lax.fori_loop(0, W // 16, zcol, 0)
    lax.fori_loop(0, C, zrow, 0)

    n_mine = (RCH - sid + NT - 1) // NT

    def zchunk(k, _):
      r = sid + k * NT
      pltpu.async_copy(rows0, acc.at[pl.ds(r * C, C)], wsem)
      return 0
    lax.fori_loop(0, n_mine, zchunk, 0)

    if split_deg:
      def zob(i, _):
        ones_buf[i, pl.ds(0, 16)] = z16
        return 0
      lax.fori_loop(0, C, zob, 0)

      def zdchunk(k, _):
        r = sid + k * NT
        pltpu.async_copy(ones_buf, acc_deg.at[pl.ds(r * C, C)], wsem)
        return 0
      lax.fori_loop(0, n_mine, zdchunk, 0)

    def zdrain(k, _):
      pltpu.make_async_copy(rows0, acc.at[pl.ds(sid * C, C)], wsem).wait()
      return 0
    lax.fori_loop(0, n_mine, zdrain, 0)

    if split_deg:
      def zddrain(k, _):
        pltpu.make_async_copy(ones_buf, acc_deg.at[pl.ds(sid * C, C)],
                              wsem).wait()
        return 0
      lax.fori_loop(0, n_mine, zddrain, 0)

      o16 = jnp.ones((16,), jnp.float32)

      def sob(i, _):
        ones_buf[i, pl.ds(0, 16)] = o16
        return 0
      lax.fori_loop(0, C, sob, 0)

    plsc.subcore_barrier()

    rows = (rows0, rows1)
    gsem = (gsem0, gsem1)
    ssem = (ssem0, ssem1)

    def run(src, dst):
      pltpu.async_copy(src.at[sid, pl.ds(0, IB)], sidx.at[0], sisem)
      pltpu.async_copy(dst.at[sid, pl.ds(0, IB)], didx.at[0], disem)

      def block(b, _):
        par = lax.rem(b, 2)
        par2 = lax.rem(b + 1, 2)
        pltpu.make_async_copy(src.at[sid, pl.ds(b * IB, IB)],
                              sidx.at[par], sisem).wait()
        pltpu.make_async_copy(dst.at[sid, pl.ds(b * IB, IB)],
                              didx.at[par], disem).wait()

        for rem in range(IB):
          buf = rem % 2
          if rem >= 2:
            pltpu.make_async_copy(
                rows[buf], acc.at[didx.at[par, rem - 2]], ssem[buf]).wait()
            if split_deg:
              pltpu.make_async_copy(
                  ones_buf, acc_deg.at[didx.at[par, rem - 2]],
                  dsem[buf]).wait()
          else:
            @pl.when(b > 0)
            def _(buf=buf, rem=rem, par2=par2):
              pltpu.make_async_copy(
                  rows[buf], acc.at[didx.at[par2, IB + rem - 2]],
                  ssem[buf]).wait()
              if split_deg:
                pltpu.make_async_copy(
                    ones_buf, acc_deg.at[didx.at[par2, IB + rem - 2]],
                    dsem[buf]).wait()
          pltpu.async_copy(x.at[sidx.at[par, rem]], rows[buf], gsem[buf])

          if rem == 3:
            @pl.when(b + 1 < NB)
            def _(par2=par2):
              pltpu.async_copy(src.at[sid, pl.ds((b + 1) * IB, IB)],
                               sidx.at[par2], sisem)
              pltpu.async_copy(dst.at[sid, pl.ds((b + 1) * IB, IB)],
                               didx.at[par2], disem)

          jbuf = 1 - buf
          if rem >= 1:
            pltpu.make_async_copy(
                x.at[sidx.at[par, rem - 1]], rows[jbuf], gsem[jbuf]).wait()
            pltpu.async_copy(rows[jbuf], acc.at[didx.at[par, rem - 1]],
                             ssem[jbuf], add=True)
            if split_deg:
              pltpu.async_copy(ones_buf, acc_deg.at[didx.at[par, rem - 1]],
                               dsem[jbuf], add=True)
          else:
            @pl.when(b > 0)
            def _(jbuf=jbuf, par2=par2):
              pltpu.make_async_copy(
                  x.at[sidx.at[par2, IB - 1]], rows[jbuf],
                  gsem[jbuf]).wait()
              pltpu.async_copy(rows[jbuf], acc.at[didx.at[par2, IB - 1]],
                               ssem[jbuf], add=True)
              if split_deg:
                pltpu.async_copy(ones_buf,
                                 acc_deg.at[didx.at[par2, IB - 1]],
                                 dsem[jbuf], add=True)
        return 0
      lax.fori_loop(0, NB, block, 0)

      lpar = (NB - 1) % 2
      pltpu.make_async_copy(x.at[sidx.at[lpar, IB - 1]], rows1,
                            gsem1).wait()
      pltpu.async_copy(rows1, acc.at[didx.at[lpar, IB - 1]], ssem1,
                       add=True)
      if split_deg:
        pltpu.async_copy(ones_buf, acc_deg.at[didx.at[lpar, IB - 1]],
                         dsem1, add=True)
      pltpu.make_async_copy(rows0, acc.at[didx.at[lpar, IB - 2]],
                            ssem0).wait()
      pltpu.make_async_copy(rows1, acc.at[didx.at[lpar, IB - 1]],
                            ssem1).wait()
      if split_deg:
        pltpu.make_async_copy(ones_buf, acc_deg.at[didx.at[lpar, IB - 2]],
                              dsem0).wait()
        pltpu.make_async_copy(ones_buf, acc_deg.at[didx.at[lpar, IB - 1]],
                              dsem1).wait()

    pl.when(cid == 0)(lambda: run(src0, dst0))
    pl.when(cid == 1)(lambda: run(src1, dst1))

    plsc.subcore_barrier()

    def wchunk(k, _):
      r = sid + k * NT
      pltpu.async_copy(acc.at[pl.ds(r * C, C)],
                       out.at[pl.ds(cid * N + r * C, C)], wsem)
      if split_deg:
        pltpu.async_copy(acc_deg.at[pl.ds(r * C, C)],
                         dout.at[pl.ds(cid * N + r * C, C)], wsem)
      return 0
    lax.fori_loop(0, n_mine, wchunk, 0)

    def wdrain(k, _):
      pltpu.make_async_copy(acc.at[pl.ds(sid * C, C)],
                            out.at[pl.ds(cid * N + sid * C, C)], wsem).wait()
      if split_deg:
        pltpu.make_async_copy(acc_deg.at[pl.ds(sid * C, C)],
                              dout.at[pl.ds(cid * N + sid * C, C)],
                              wsem).wait()
      return 0
    lax.fori_loop(0, n_mine, wdrain, 0)

  return agg


_sc_agg_deg = _make_sc_agg(D, split_deg=True)
_sc_agg_d = _make_sc_agg(D)


def _layer1_body(agg_ref, deg_ref, x_ref, wl_ref, bl_ref, wr_ref, o_ref):
  deg = jnp.maximum(deg_ref[...], 1.0)
  mean = agg_ref[...] / deg
  h = lax.dot_general(mean, wl_ref[...], (((1,), (1,)), ((), ())),
                      preferred_element_type=jnp.float32)
  h = h + bl_ref[...]
  h = h + lax.dot_general(x_ref[:, :D], wr_ref[...], (((1,), (1,)), ((), ())),
                          preferred_element_type=jnp.float32)
  o_ref[...] = jnp.maximum(h, 0.0)


def _layer2_body(agg_ref, deg_ref, h_ref, wl_ref, bl_ref, wr_ref, o_ref):
  deg = jnp.maximum(deg_ref[...], 1.0)
  mean = agg_ref[...] / deg
  g = lax.dot_general(mean, wl_ref[...], (((1,), (1,)), ((), ())),
                      preferred_element_type=jnp.float32)
  g = g + bl_ref[...]
  g = g + lax.dot_general(h_ref[...], wr_ref[...], (((1,), (1,)), ((), ())),
                          preferred_element_type=jnp.float32)
  nrm = jnp.sqrt(jnp.sum(g * g, axis=1, keepdims=True))
  o_ref[...] = g / jnp.maximum(nrm, 1e-12)


def _tc_layer1(agg, deg, xp, wl, bl, wr):
  m = agg.shape[0]
  grid = m // ROWS_BLK
  return pl.pallas_call(
      _layer1_body,
      grid=(grid,),
      in_specs=[
          pl.BlockSpec((ROWS_BLK, D), lambda i: (i, 0)),
          pl.BlockSpec((ROWS_BLK, 1), lambda i: (i, 0)),
          pl.BlockSpec((ROWS_BLK, D), lambda i: (i, 0)),
          pl.BlockSpec((D, D), lambda i: (0, 0)),
          pl.BlockSpec((1, D), lambda i: (0, 0)),
          pl.BlockSpec((D, D), lambda i: (0, 0)),
      ],
      out_specs=pl.BlockSpec((ROWS_BLK, D), lambda i: (i, 0)),
      out_shape=jax.ShapeDtypeStruct((m, D), jnp.float32),
  )(agg, deg, xp, wl, bl, wr)


def _tc_layer2(agg, deg, h, wl, bl, wr):
  m = agg.shape[0]
  grid = m // ROWS_BLK
  return pl.pallas_call(
      _layer2_body,
      grid=(grid,),
      in_specs=[
          pl.BlockSpec((ROWS_BLK, D), lambda i: (i, 0)),
          pl.BlockSpec((ROWS_BLK, 1), lambda i: (i, 0)),
          pl.BlockSpec((ROWS_BLK, D), lambda i: (i, 0)),
          pl.BlockSpec((D, D), lambda i: (0, 0)),
          pl.BlockSpec((1, D), lambda i: (0, 0)),
          pl.BlockSpec((D, D), lambda i: (0, 0)),
      ],
      out_specs=pl.BlockSpec((ROWS_BLK, D), lambda i: (i, 0)),
      out_shape=jax.ShapeDtypeStruct((m, D), jnp.float32),
  )(agg, deg, h, wl, bl, wr)


def kernel(x_orig, edge_index_orig, x_anon, edge_index_anon,
           Wl1, bl1, Wr1, Wl2, bl2, Wr2):
  src_o = edge_index_orig[0].astype(jnp.int32).reshape(NT, NB * IB, C)
  dst_o = edge_index_orig[1].astype(jnp.int32).reshape(NT, NB * IB, C)
  src_a = (edge_index_anon[0].astype(jnp.int32) + N).reshape(NT, NB * IB, C)
  dst_a = edge_index_anon[1].astype(jnp.int32).reshape(NT, NB * IB, C)

  x2 = jnp.concatenate([x_orig, x_anon], axis=0)

  agg1, degw = _sc_agg_deg(x2, src_o, dst_o, src_a, dst_a)
  deg = lax.slice(degw, (0, 0), (2 * N, 1))
  h = _tc_layer1(agg1, deg, x2, Wl1, bl1.reshape(1, D), Wr1)
  agg2 = _sc_agg_d(h, src_o, dst_o, src_a, dst_a)
  out = _tc_layer2(agg2, deg, h, Wl2, bl2.reshape(1, D), Wr2)
  return (out[:N], out[N:])


# speedup vs baseline: 9.9562x; 1.0001x over previous
import functools

import jax
import jax.numpy as jnp
from jax import lax
from jax.experimental import pallas as pl
from jax.experimental.pallas import tpu as pltpu
from jax.experimental.pallas import tpu_sc as plsc

N = 10000
D = 128
E = 320000
NT = 16
C = 80
EPT = E // NT
NCHUNK = EPT // C
IB = 10
NB = NCHUNK // IB
RCH = N // C
ROWS_BLK = 2000


def _make_sc_agg(W, split_deg=False):
  mesh = plsc.VectorSubcoreMesh(core_axis_name="c", subcore_axis_name="s")

  DW = 16
  if split_deg:
    out_type = (jax.ShapeDtypeStruct((2 * N, W), jnp.float32),
                jax.ShapeDtypeStruct((2 * N, DW), jnp.float32))
  else:
    out_type = jax.ShapeDtypeStruct((2 * N, W), jnp.float32)

  @functools.partial(
      pl.kernel,
      mesh=mesh,
      out_type=out_type,
      scratch_types=[
          pltpu.VMEM((2, IB, C), jnp.int32),
          pltpu.VMEM((2, IB, C), jnp.int32),
          pltpu.VMEM((C, W), jnp.float32),
          pltpu.VMEM((C, W), jnp.float32),
          pltpu.VMEM_SHARED((N, W), jnp.float32),
          pltpu.SemaphoreType.DMA,
          pltpu.SemaphoreType.DMA,
          pltpu.SemaphoreType.DMA,
          pltpu.SemaphoreType.DMA,
          pltpu.SemaphoreType.DMA,
          pltpu.SemaphoreType.DMA,
          pltpu.SemaphoreType.DMA,
      ] + ([
          pltpu.VMEM((C, DW), jnp.float32),
          pltpu.VMEM_SHARED((N, DW), jnp.float32),
          pltpu.SemaphoreType.DMA,
          pltpu.SemaphoreType.DMA,
      ] if split_deg else []),
      compiler_params=pltpu.CompilerParams(use_tc_tiling_on_sc=False),
  )
  def agg(x, src0, dst0, src1, dst1, *rest):
    if split_deg:
      (out, dout, sidx, didx, rows0, rows1, acc,
       gsem0, gsem1, ssem0, ssem1, sisem, disem, wsem,
       ones_buf, acc_deg, dsem0, dsem1) = rest
      dsem = (dsem0, dsem1)
    else:
      (out, sidx, didx, rows0, rows1, acc,
       gsem0, gsem1, ssem0, ssem1, sisem, disem, wsem) = rest
    cid = lax.axis_index("c")
    sid = lax.axis_index("s")

    z16 = jnp.zeros((16,), jnp.float32)

    def zrow(i, _):
      def zcol(j, _):
        rows0[i, pl.ds(j * 16, 16)] = z16
        return 0
      return lax.fori_loop(0, W // 16, zcol, 0)
    lax.fori_loop(0, C, zrow, 0)

    n_mine = (RCH - sid + NT - 1) // NT

    def zchunk(k, _):
      r = sid + k * NT
      pltpu.async_copy(rows0, acc.at[pl.ds(r * C, C)], wsem)
      return 0
    lax.fori_loop(0, n_mine, zchunk, 0)

    if split_deg:
      def zob(i, _):
        ones_buf[i, pl.ds(0, 16)] = z16
        return 0
      lax.fori_loop(0, C, zob, 0)

      def zdchunk(k, _):
        r = sid + k * NT
        pltpu.async_copy(ones_buf, acc_deg.at[pl.ds(r * C, C)], wsem)
        return 0
      lax.fori_loop(0, n_mine, zdchunk, 0)

    def zdrain(k, _):
      pltpu.make_async_copy(rows0, acc.at[pl.ds(sid * C, C)], wsem).wait()
      return 0
    lax.fori_loop(0, n_mine, zdrain, 0)

    if split_deg:
      def zddrain(k, _):
        pltpu.make_async_copy(ones_buf, acc_deg.at[pl.ds(sid * C, C)],
                              wsem).wait()
        return 0
      lax.fori_loop(0, n_mine, zddrain, 0)

      o16 = jnp.ones((16,), jnp.float32)

      def sob(i, _):
        ones_buf[i, pl.ds(0, 16)] = o16
        return 0
      lax.fori_loop(0, C, sob, 0)

    plsc.subcore_barrier()

    rows = (rows0, rows1)
    gsem = (gsem0, gsem1)
    ssem = (ssem0, ssem1)

    def run(src, dst):
      pltpu.async_copy(src.at[sid, pl.ds(0, IB)], sidx.at[0], sisem)
      pltpu.async_copy(dst.at[sid, pl.ds(0, IB)], didx.at[0], disem)

      def block(b, _):
        par = lax.rem(b, 2)
        par2 = lax.rem(b + 1, 2)
        pltpu.make_async_copy(src.at[sid, pl.ds(b * IB, IB)],
                              sidx.at[par], sisem).wait()
        pltpu.make_async_copy(dst.at[sid, pl.ds(b * IB, IB)],
                              didx.at[par], disem).wait()

        for rem in range(IB):
          buf = rem % 2
          if rem >= 2:
            pltpu.make_async_copy(
                rows[buf], acc.at[didx.at[par, rem - 2]], ssem[buf]).wait()
            if split_deg:
              pltpu.make_async_copy(
                  ones_buf, acc_deg.at[didx.at[par, rem - 2]],
                  dsem[buf]).wait()
          else:
            @pl.when(b > 0)
            def _(buf=buf, rem=rem, par2=par2):
              pltpu.make_async_copy(
                  rows[buf], acc.at[didx.at[par2, IB + rem - 2]],
                  ssem[buf]).wait()
              if split_deg:
                pltpu.make_async_copy(
                    ones_buf, acc_deg.at[didx.at[par2, IB + rem - 2]],
                    dsem[buf]).wait()
          pltpu.async_copy(x.at[sidx.at[par, rem]], rows[buf], gsem[buf])

          if rem == 3:
            @pl.when(b + 1 < NB)
            def _(par2=par2):
              pltpu.async_copy(src.at[sid, pl.ds((b + 1) * IB, IB)],
                               sidx.at[par2], sisem)
              pltpu.async_copy(dst.at[sid, pl.ds((b + 1) * IB, IB)],
                               didx.at[par2], disem)

          jbuf = 1 - buf
          if rem >= 1:
            pltpu.make_async_copy(
                x.at[sidx.at[par, rem - 1]], rows[jbuf], gsem[jbuf]).wait()
            pltpu.async_copy(rows[jbuf], acc.at[didx.at[par, rem - 1]],
                             ssem[jbuf], add=True)
            if split_deg:
              pltpu.async_copy(ones_buf, acc_deg.at[didx.at[par, rem - 1]],
                               dsem[jbuf], add=True)
          else:
            @pl.when(b > 0)
            def _(jbuf=jbuf, par2=par2):
              pltpu.make_async_copy(
                  x.at[sidx.at[par2, IB - 1]], rows[jbuf],
                  gsem[jbuf]).wait()
              pltpu.async_copy(rows[jbuf], acc.at[didx.at[par2, IB - 1]],
                               ssem[jbuf], add=True)
              if split_deg:
                pltpu.async_copy(ones_buf,
                                 acc_deg.at[didx.at[par2, IB - 1]],
                                 dsem[jbuf], add=True)
        return 0
      lax.fori_loop(0, NB, block, 0)

      lpar = (NB - 1) % 2
      pltpu.make_async_copy(x.at[sidx.at[lpar, IB - 1]], rows1,
                            gsem1).wait()
      pltpu.async_copy(rows1, acc.at[didx.at[lpar, IB - 1]], ssem1,
                       add=True)
      if split_deg:
        pltpu.async_copy(ones_buf, acc_deg.at[didx.at[lpar, IB - 1]],
                         dsem1, add=True)
      pltpu.make_async_copy(rows0, acc.at[didx.at[lpar, IB - 2]],
                            ssem0).wait()
      pltpu.make_async_copy(rows1, acc.at[didx.at[lpar, IB - 1]],
                            ssem1).wait()
      if split_deg:
        pltpu.make_async_copy(ones_buf, acc_deg.at[didx.at[lpar, IB - 2]],
                              dsem0).wait()
        pltpu.make_async_copy(ones_buf, acc_deg.at[didx.at[lpar, IB - 1]],
                              dsem1).wait()

    pl.when(cid == 0)(lambda: run(src0, dst0))
    pl.when(cid == 1)(lambda: run(src1, dst1))

    plsc.subcore_barrier()

    def wchunk(k, _):
      r = sid + k * NT
      pltpu.async_copy(acc.at[pl.ds(r * C, C)],
                       out.at[pl.ds(cid * N + r * C, C)], wsem)
      if split_deg:
        pltpu.async_copy(acc_deg.at[pl.ds(r * C, C)],
                         dout.at[pl.ds(cid * N + r * C, C)], wsem)
      return 0
    lax.fori_loop(0, n_mine, wchunk, 0)

    def wdrain(k, _):
      pltpu.make_async_copy(acc.at[pl.ds(sid * C, C)],
                            out.at[pl.ds(cid * N + sid * C, C)], wsem).wait()
      if split_deg:
        pltpu.make_async_copy(acc_deg.at[pl.ds(sid * C, C)],
                              dout.at[pl.ds(cid * N + sid * C, C)],
                              wsem).wait()
      return 0
    lax.fori_loop(0, n_mine, wdrain, 0)

  return agg


_sc_agg_deg = _make_sc_agg(D, split_deg=True)
_sc_agg_d = _make_sc_agg(D)


def _layer1_body(agg_ref, deg_ref, x_ref, wl_ref, bl_ref, wr_ref, o_ref):
  deg = jnp.maximum(deg_ref[...], 1.0)
  mean = agg_ref[...] / deg
  h = lax.dot_general(mean, wl_ref[...], (((1,), (1,)), ((), ())),
                      preferred_element_type=jnp.float32)
  h = h + bl_ref[...]
  h = h + lax.dot_general(x_ref[:, :D], wr_ref[...], (((1,), (1,)), ((), ())),
                          preferred_element_type=jnp.float32)
  o_ref[...] = jnp.maximum(h, 0.0)


def _layer2_body(agg_ref, deg_ref, h_ref, wl_ref, bl_ref, wr_ref, o_ref):
  deg = jnp.maximum(deg_ref[...], 1.0)
  mean = agg_ref[...] / deg
  g = lax.dot_general(mean, wl_ref[...], (((1,), (1,)), ((), ())),
                      preferred_element_type=jnp.float32)
  g = g + bl_ref[...]
  g = g + lax.dot_general(h_ref[...], wr_ref[...], (((1,), (1,)), ((), ())),
                          preferred_element_type=jnp.float32)
  nrm = jnp.sqrt(jnp.sum(g * g, axis=1, keepdims=True))
  o_ref[...] = g / jnp.maximum(nrm, 1e-12)


def _tc_layer1(agg, deg, xp, wl, bl, wr):
  m = agg.shape[0]
  grid = m // ROWS_BLK
  return pl.pallas_call(
      _layer1_body,
      grid=(grid,),
      in_specs=[
          pl.BlockSpec((ROWS_BLK, D), lambda i: (i, 0)),
          pl.BlockSpec((ROWS_BLK, 1), lambda i: (i, 0)),
          pl.BlockSpec((ROWS_BLK, D), lambda i: (i, 0)),
          pl.BlockSpec((D, D), lambda i: (0, 0)),
          pl.BlockSpec((1, D), lambda i: (0, 0)),
          pl.BlockSpec((D, D), lambda i: (0, 0)),
      ],
      out_specs=pl.BlockSpec((ROWS_BLK, D), lambda i: (i, 0)),
      out_shape=jax.ShapeDtypeStruct((m, D), jnp.float32),
  )(agg, deg, xp, wl, bl, wr)


def _tc_layer2(agg, deg, h, wl, bl, wr):
  m = agg.shape[0]
  grid = m // ROWS_BLK
  return pl.pallas_call(
      _layer2_body,
      grid=(grid,),
      in_specs=[
          pl.BlockSpec((ROWS_BLK, D), lambda i: (i, 0)),
          pl.BlockSpec((ROWS_BLK, 1), lambda i: (i, 0)),
          pl.BlockSpec((ROWS_BLK, D), lambda i: (i, 0)),
          pl.BlockSpec((D, D), lambda i: (0, 0)),
          pl.BlockSpec((1, D), lambda i: (0, 0)),
          pl.BlockSpec((D, D), lambda i: (0, 0)),
      ],
      out_specs=pl.BlockSpec((ROWS_BLK, D), lambda i: (i, 0)),
      out_shape=jax.ShapeDtypeStruct((m, D), jnp.float32),
  )(agg, deg, h, wl, bl, wr)


def kernel(x_orig, edge_index_orig, x_anon, edge_index_anon,
           Wl1, bl1, Wr1, Wl2, bl2, Wr2):
  src_o = edge_index_orig[0].astype(jnp.int32).reshape(NT, NB * IB, C)
  dst_o = edge_index_orig[1].astype(jnp.int32).reshape(NT, NB * IB, C)
  src_a = (edge_index_anon[0].astype(jnp.int32) + N).reshape(NT, NB * IB, C)
  dst_a = edge_index_anon[1].astype(jnp.int32).reshape(NT, NB * IB, C)

  x2 = jnp.concatenate([x_orig, x_anon], axis=0)

  agg1, degw = _sc_agg_deg(x2, src_o, dst_o, src_a, dst_a)
  deg = lax.slice(degw, (0, 0), (2 * N, 1))
  h = _tc_layer1(agg1, deg, x2, Wl1, bl1.reshape(1, D), Wr1)
  agg2 = _sc_agg_d(h, src_o, dst_o, src_a, dst_a)
  out = _tc_layer2(agg2, deg, h, Wl2, bl2.reshape(1, D), Wr2)
  return (out[:N], out[N:])
